# Initial kernel scaffold; baseline (speedup 1.0000x reference)
import jax
import jax.numpy as jnp
from jax.experimental import pallas as pl


def kernel(x, edge_index, batch, W1, att1, W2, att2, W3, att3, lin1_w, lin1_b, lin2_w, lin2_b, lin3_w, lin3_b):
    raise NotImplementedError("write your pallas kernel here")


# speedup vs baseline: 1.6191x; 1.6191x over previous
import functools

import jax
import jax.numpy as jnp
from jax import lax
from jax.experimental import pallas as pl
from jax.experimental.pallas import tpu as pltpu

N = 10000
B = 64
NHID = 512
HEADS = 4
RATIO = 0.5

_BN = 400


def _dense_body(xagg_ref, x_ref, w_ref, att_t_ref, deg_ref, mask_ref,
                out_ref, key_ref, ts_ref):
    xin = xagg_ref[...] + x_ref[...]
    y = jnp.dot(xin, w_ref[...], preferred_element_type=jnp.float32)
    deg = jnp.maximum(deg_ref[...], 1.0)
    out = y / deg
    out_ref[...] = out
    sh = jnp.tanh(jnp.dot(out, att_t_ref[...],
                          preferred_element_type=jnp.float32))
    s = jnp.mean(sh, axis=1, keepdims=True)
    mask = mask_ref[...]
    key_ref[...] = jnp.where(mask > 0, s, -1e9)
    ts_ref[...] = jnp.tanh(s)


def _dense_layer(xagg, x, w, att, deg, mask):
    d = x.shape[1]
    att_t = att.T
    grid = (N // _BN,)
    return pl.pallas_call(
        _dense_body,
        grid=grid,
        in_specs=[
            pl.BlockSpec((_BN, d), lambda i: (i, 0)),
            pl.BlockSpec((_BN, d), lambda i: (i, 0)),
            pl.BlockSpec((d, NHID), lambda i: (0, 0)),
            pl.BlockSpec((NHID, HEADS), lambda i: (0, 0)),
            pl.BlockSpec((_BN, 1), lambda i: (i, 0)),
            pl.BlockSpec((_BN, 1), lambda i: (i, 0)),
        ],
        out_specs=[
            pl.BlockSpec((_BN, NHID), lambda i: (i, 0)),
            pl.BlockSpec((_BN, 1), lambda i: (i, 0)),
            pl.BlockSpec((_BN, 1), lambda i: (i, 0)),
        ],
        out_shape=[
            jax.ShapeDtypeStruct((N, NHID), jnp.float32),
            jax.ShapeDtypeStruct((N, 1), jnp.float32),
            jax.ShapeDtypeStruct((N, 1), jnp.float32),
        ],
    )(xagg, x, w, att_t, deg[:, None], mask[:, None])


def _keep_mask(key, batch, k_per):
    n = key.shape[0]
    order = jnp.lexsort((-key, batch))
    counts_all = jnp.bincount(batch, length=B)
    offsets = jnp.concatenate(
        [jnp.zeros((1,), counts_all.dtype), jnp.cumsum(counts_all)[:-1]])
    rank_sorted = jnp.arange(n) - offsets[batch[order]]
    rank = jnp.zeros((n,), rank_sorted.dtype).at[order].set(rank_sorted)
    keep = (rank < k_per[batch]) & (key > -1e8)
    return keep.astype(jnp.float32)


def _readout(x, batch, mask):
    d = x.shape[1]
    ssum = jnp.zeros((B, d), x.dtype).at[batch].add(x)
    cnt_raw = jnp.zeros((B,), x.dtype).at[batch].add(mask)
    cnt = jnp.maximum(cnt_raw, 1.0)
    mean = ssum / cnt[:, None]
    xm = jnp.where(mask[:, None] > 0, x, -1e30)
    mx = jnp.full((B, d), -1e30, x.dtype).at[batch].max(xm)
    mx = jnp.where(mx <= -1e29, 0.0, mx)
    return jnp.concatenate([ssum, mean, mx], axis=-1), cnt_raw


def _edge_agg(x, mask, src, dst):
    xagg = jnp.zeros_like(x).at[dst].add(x[src])
    deg = jnp.zeros((N,), x.dtype).at[dst].add(mask[src]) + mask
    return xagg, deg


def _elsa_layer(x, mask, src, dst, batch, w, att, n_active, use_relu):
    xagg, deg = _edge_agg(x, mask, src, dst)
    out, key, ts = _dense_layer(xagg, x, w, att, deg, mask)
    k_per = jnp.ceil(RATIO * n_active)
    new_mask = _keep_mask(key[:, 0], batch, k_per)
    x_out = out * ts * new_mask[:, None]
    if use_relu:
        x_out = jax.nn.relu(x_out)
    return x_out, new_mask


def kernel(x, edge_index, batch, W1, att1, W2, att2, W3, att3,
           lin1_w, lin1_b, lin2_w, lin2_b, lin3_w, lin3_b):
    src, dst = edge_index[0], edge_index[1]
    mask0 = jnp.ones((N,), x.dtype)
    n_active0 = jnp.bincount(batch, length=B).astype(x.dtype)

    x1, m1 = _elsa_layer(x, mask0, src, dst, batch, W1, att1, n_active0,
                         use_relu=False)
    xs, cnt1 = _readout(x1, batch, m1)
    x2, m2 = _elsa_layer(x1, m1, src, dst, batch, W2, att2, cnt1,
                         use_relu=True)
    r2, cnt2 = _readout(x2, batch, m2)
    xs = xs + r2
    x3, m3 = _elsa_layer(x2, m2, src, dst, batch, W3, att3, cnt2,
                         use_relu=True)
    r3, _ = _readout(x3, batch, m3)
    xs = xs + r3

    h = jax.nn.relu(xs @ lin1_w + lin1_b)
    h = jax.nn.relu(h @ lin2_w + lin2_b)
    logits = h @ lin3_w + lin3_b
    return jax.nn.log_softmax(logits, axis=-1)


# speedup vs baseline: 2.1490x; 1.3273x over previous
import functools

import jax
import jax.numpy as jnp
from jax import lax
from jax.experimental import pallas as pl
from jax.experimental.pallas import tpu as pltpu
from jax.experimental.pallas import tpu_sc as plsc

N = 10000
B = 64
NHID = 512
HEADS = 4
RATIO = 0.5

_BN = 400

_CC = 64
_EK = 80
_NTILES = 16
_NPAD = 10240
_EPAD = 163840
_NK = _EPAD // _NTILES // _EK
_DUMMY_DST = N


def _make_agg_kernel(nc):
    cpc = nc // 2
    mesh = plsc.VectorSubcoreMesh(core_axis_name="c", subcore_axis_name="s")

    def body(*refs):
        xcs = refs[0:nc]
        srcr = refs[nc]
        dstr = refs[nc + 1]
        zrows = refs[nc + 2]
        outs = refs[nc + 3:nc + 3 + nc]
        (src_ids, dst_ids, bufs, acc, sem0, sem1) = refs[nc + 3 + nc:]
        sems = (sem0, sem1)

        cid = lax.axis_index("c")
        sid = lax.axis_index("s")
        row0 = sid * (_NPAD // _NTILES)
        nrow = _NPAD // _NTILES

        pltpu.sync_copy(srcr.at[sid], src_ids)
        pltpu.sync_copy(dstr.at[sid], dst_ids)

        def run_phase(xch, oh):
            pltpu.sync_copy(zrows, acc.at[pl.ds(row0, nrow)])
            plsc.subcore_barrier()
            for b in range(2):
                pltpu.make_async_copy(
                    xch.at[src_ids.at[b]], bufs.at[b], sems[b]).start()

            def step(i, carry):
                for b in range(2):
                    k = 2 * i + b
                    pltpu.make_async_copy(
                        xch.at[src_ids.at[k]], bufs.at[b], sems[b]).wait()
                    pltpu.sync_copy(bufs.at[b],
                                    acc.at[dst_ids.at[k]], add=True)

                    @pl.when(k + 2 < _NK)
                    def _():
                        pltpu.make_async_copy(
                            xch.at[src_ids.at[k + 2]], bufs.at[b],
                            sems[b]).start()
                return carry

            lax.fori_loop(0, _NK // 2, step, 0)
            plsc.subcore_barrier()
            pltpu.sync_copy(acc.at[pl.ds(row0, nrow)],
                            oh.at[pl.ds(row0, nrow)])

        for core_val in range(2):
            @pl.when(cid == core_val)
            def _():
                for p in range(cpc):
                    ch = core_val * cpc + p
                    run_phase(xcs[ch], outs[ch])

        return None

    kern = pl.kernel(
        body,
        mesh=mesh,
        compiler_params=pltpu.CompilerParams(use_tc_tiling_on_sc=False),
        out_type=[jax.ShapeDtypeStruct((_NPAD, _CC), jnp.float32)
                  for _ in range(nc)],
        scratch_types=[
            pltpu.VMEM((_NK, _EK), jnp.int32),
            pltpu.VMEM((_NK, _EK), jnp.int32),
            pltpu.VMEM((2, _EK, _CC), jnp.float32),
            pltpu.VMEM_SHARED((_NPAD, _CC), jnp.float32),
            pltpu.SemaphoreType.DMA,
            pltpu.SemaphoreType.DMA,
        ],
    )
    return kern


@functools.partial(jax.jit, static_argnames=("nc",))
def _sc_edge_agg(x, srcr, dstr, nc):
    xcs = [x[:, i * _CC:(i + 1) * _CC] for i in range(nc)]
    zrows = jnp.zeros((_NPAD // _NTILES, _CC), jnp.float32)
    outs = _make_agg_kernel(nc)(*xcs, srcr, dstr, zrows)
    return jnp.concatenate([o[:N] for o in outs], axis=1)


def _dense_body(xagg_ref, x_ref, w_ref, att_t_ref, deg_ref, mask_ref,
                out_ref, key_ref, ts_ref):
    xin = xagg_ref[...] + x_ref[...]
    y = jnp.dot(xin, w_ref[...], preferred_element_type=jnp.float32)
    deg = jnp.maximum(deg_ref[...], 1.0)
    out = y / deg
    out_ref[...] = out
    sh = jnp.tanh(jnp.dot(out, att_t_ref[...],
                          preferred_element_type=jnp.float32))
    s = jnp.mean(sh, axis=1, keepdims=True)
    mask = mask_ref[...]
    key_ref[...] = jnp.where(mask > 0, s, -1e9)
    ts_ref[...] = jnp.tanh(s)


def _dense_layer(xagg, x, w, att, deg, mask):
    d = x.shape[1]
    att_t = att.T
    grid = (N // _BN,)
    return pl.pallas_call(
        _dense_body,
        grid=grid,
        in_specs=[
            pl.BlockSpec((_BN, d), lambda i: (i, 0)),
            pl.BlockSpec((_BN, d), lambda i: (i, 0)),
            pl.BlockSpec((d, NHID), lambda i: (0, 0)),
            pl.BlockSpec((NHID, HEADS), lambda i: (0, 0)),
            pl.BlockSpec((_BN, 1), lambda i: (i, 0)),
            pl.BlockSpec((_BN, 1), lambda i: (i, 0)),
        ],
        out_specs=[
            pl.BlockSpec((_BN, NHID), lambda i: (i, 0)),
            pl.BlockSpec((_BN, 1), lambda i: (i, 0)),
            pl.BlockSpec((_BN, 1), lambda i: (i, 0)),
        ],
        out_shape=[
            jax.ShapeDtypeStruct((N, NHID), jnp.float32),
            jax.ShapeDtypeStruct((N, 1), jnp.float32),
            jax.ShapeDtypeStruct((N, 1), jnp.float32),
        ],
    )(xagg, x, w, att_t, deg[:, None], mask[:, None])


def _keep_mask(key, batch, k_per):
    n = key.shape[0]
    order = jnp.lexsort((-key, batch))
    counts_all = jnp.bincount(batch, length=B)
    offsets = jnp.concatenate(
        [jnp.zeros((1,), counts_all.dtype), jnp.cumsum(counts_all)[:-1]])
    rank_sorted = jnp.arange(n) - offsets[batch[order]]
    rank = jnp.zeros((n,), rank_sorted.dtype).at[order].set(rank_sorted)
    keep = (rank < k_per[batch]) & (key > -1e8)
    return keep.astype(jnp.float32)


def _readout(x, batch, mask):
    d = x.shape[1]
    ssum = jnp.zeros((B, d), x.dtype).at[batch].add(x)
    cnt_raw = jnp.zeros((B,), x.dtype).at[batch].add(mask)
    cnt = jnp.maximum(cnt_raw, 1.0)
    mean = ssum / cnt[:, None]
    xm = jnp.where(mask[:, None] > 0, x, -1e30)
    mx = jnp.full((B, d), -1e30, x.dtype).at[batch].max(xm)
    mx = jnp.where(mx <= -1e29, 0.0, mx)
    return jnp.concatenate([ssum, mean, mx], axis=-1), cnt_raw


def _edge_agg(x, mask, src, dst, srcr, dstr):
    xagg = _sc_edge_agg(x, srcr, dstr, nc=x.shape[1] // _CC)
    deg = jnp.zeros((N,), x.dtype).at[dst].add(mask[src]) + mask
    return xagg, deg


def _elsa_layer(x, mask, src, dst, srcr, dstr, batch, w, att, n_active,
                use_relu):
    xagg, deg = _edge_agg(x, mask, src, dst, srcr, dstr)
    out, key, ts = _dense_layer(xagg, x, w, att, deg, mask)
    k_per = jnp.ceil(RATIO * n_active)
    new_mask = _keep_mask(key[:, 0], batch, k_per)
    x_out = out * ts * new_mask[:, None]
    if use_relu:
        x_out = jax.nn.relu(x_out)
    return x_out, new_mask


def kernel(x, edge_index, batch, W1, att1, W2, att2, W3, att3,
           lin1_w, lin1_b, lin2_w, lin2_b, lin3_w, lin3_b):
    src, dst = edge_index[0], edge_index[1]
    e = src.shape[0]
    pad = _EPAD - e
    srcr = jnp.concatenate([src, jnp.zeros((pad,), src.dtype)]
                           ).reshape(_NTILES, _NK, _EK)
    dstr = jnp.concatenate([dst, jnp.full((pad,), _DUMMY_DST, dst.dtype)]
                           ).reshape(_NTILES, _NK, _EK)
    mask0 = jnp.ones((N,), x.dtype)
    n_active0 = jnp.bincount(batch, length=B).astype(x.dtype)

    x1, m1 = _elsa_layer(x, mask0, src, dst, srcr, dstr, batch, W1, att1,
                         n_active0, use_relu=False)
    xs, cnt1 = _readout(x1, batch, m1)
    x2, m2 = _elsa_layer(x1, m1, src, dst, srcr, dstr, batch, W2, att2,
                         cnt1, use_relu=True)
    r2, cnt2 = _readout(x2, batch, m2)
    xs = xs + r2
    x3, m3 = _elsa_layer(x2, m2, src, dst, srcr, dstr, batch, W3, att3,
                         cnt2, use_relu=True)
    r3, _ = _readout(x3, batch, m3)
    xs = xs + r3

    h = jax.nn.relu(xs @ lin1_w + lin1_b)
    h = jax.nn.relu(h @ lin2_w + lin2_b)
    logits = h @ lin3_w + lin3_b
    return jax.nn.log_softmax(logits, axis=-1)
